# baseline (speedup 1.0000x reference)
import functools

import jax
import jax.numpy as jnp
from jax import lax
from jax.experimental import pallas as pl
from jax.experimental.pallas import tpu as pltpu
from jax.experimental.pallas import tpu_sc as plsc

N_NODES = 10000
N_EDGES = 320000
D = 128

NP = 10240
NT = 32
CH = 128
CPT0 = 90
CPT1 = 67
CPTC = 90
E0 = 16 * CPT0 * CH
E1 = 16 * CPT1 * CH
EP = E0 + E1
RPT = NP // 16

_mesh = plsc.VectorSubcoreMesh(core_axis_name="c", subcore_axis_name="s")
_sc_params = pltpu.CompilerParams(
    needs_layout_passes=False, use_tc_tiling_on_sc=False)


@functools.partial(
    pl.kernel,
    out_type=jax.ShapeDtypeStruct((NT, NP), jnp.float32),
    mesh=_mesh,
    scratch_types=[
        pltpu.VMEM((CPTC, CH), jnp.int32),
        pltpu.VMEM((CPTC, CH), jnp.float32),
        pltpu.VMEM((NP,), jnp.float32),
    ],
    compiler_params=_sc_params,
)
def _deg_kernel(sd_hbm, w_hbm, out_hbm, sd_v, w_v, deg_v):
    c = lax.axis_index("c")
    s = lax.axis_index("s")
    wid = c * 16 + s
    my_cpt = jnp.where(c == 0, CPT0, CPT1)
    pltpu.sync_copy(sd_hbm.at[wid], sd_v)
    pltpu.sync_copy(w_hbm.at[wid], w_v)

    zero = jnp.zeros((16,), jnp.float32)

    @pl.loop(0, NP // 16)
    def _(i):
        deg_v[pl.ds(i * 16, 16)] = zero

    @pl.loop(0, my_cpt)
    def _(j):
        for q in range(CH // 16):
            idx = jnp.bitwise_and(sd_v[j, pl.ds(q * 16, 16)], 16383)
            val = w_v[j, pl.ds(q * 16, 16)]
            plsc.addupdate_scatter(deg_v, [idx], val)

    pltpu.sync_copy(deg_v, out_hbm.at[wid])


@functools.partial(
    pl.kernel,
    out_type=jax.ShapeDtypeStruct((2, NP, D), jnp.float32),
    mesh=_mesh,
    scratch_types=[
        pltpu.VMEM((CPTC, CH), jnp.int32),
        pltpu.VMEM((CPTC, CH), jnp.float32),
        pltpu.VMEM((CH,), jnp.int32),
        pltpu.VMEM((CH,), jnp.int32),
        pltpu.VMEM((CH, D), jnp.bfloat16),
        pltpu.VMEM((CH, D), jnp.float32),
        pltpu.VMEM_SHARED((NP, D), jnp.float32),
        pltpu.SemaphoreType.DMA,
    ],
    compiler_params=_sc_params,
)
def _msgpass_kernel(hs_hbm, sd_hbm, w_hbm, out_hbm,
                    sd_v, w_v, srci_v, dsti_v, rowsb_v, rows_v, acc_sh, gsem):
    c = lax.axis_index("c")
    s = lax.axis_index("s")
    wid = c * 16 + s
    my_cpt = jnp.where(c == 0, CPT0, CPT1)
    pltpu.sync_copy(sd_hbm.at[wid], sd_v)
    pltpu.sync_copy(w_hbm.at[wid], w_v)

    zero = jnp.zeros((16,), jnp.float32)

    @pl.loop(0, CH)
    def _(r):
        for q in range(D // 16):
            rows_v[r, pl.ds(q * 16, 16)] = zero

    base = s * RPT
    for k in range(RPT // CH):
        pltpu.sync_copy(rows_v, acc_sh.at[pl.ds(base + k * CH, CH)])
    plsc.subcore_barrier()

    @pl.loop(0, my_cpt)
    def _(j):
        @plsc.parallel_loop(0, CH // 16)
        def _(q):
            sd = sd_v[j, pl.ds(q * 16, 16)]
            srci_v[pl.ds(q * 16, 16)] = lax.shift_right_logical(sd, 14)
            dsti_v[pl.ds(q * 16, 16)] = jnp.bitwise_and(sd, 16383)

        pltpu.async_copy(hs_hbm.at[srci_v], rowsb_v, gsem).wait()

        @plsc.parallel_loop(0, CH // 16)
        def _(g):
            wvec = w_v[j, pl.ds(g * 16, 16)]
            for r in range(16):
                wv = wvec[r]
                row = g * 16 + r
                for q in range(D // 32):
                    ab = rowsb_v[row, pl.ds(q * 32, 32)]
                    a, b = plsc.unpack(ab, format=plsc.PackFormat.INTERLEAVED)
                    rows_v[row, pl.ds(q * 32, 16)] = a * wv
                    rows_v[row, pl.ds(q * 32 + 16, 16)] = b * wv

        pltpu.sync_copy(rows_v, acc_sh.at[dsti_v], add=True)

    plsc.subcore_barrier()
    pltpu.sync_copy(acc_sh.at[pl.ds(s * RPT, RPT)],
                    out_hbm.at[c, pl.ds(s * RPT, RPT)])


_GRID = NP // 1024


def _tc_a_body(x_ref, w1_ref, degp_ref, h1_ref, hs1_ref, dinv_ref):
    deg = 1.0 + jnp.sum(degp_ref[...], axis=0)
    dinv = lax.rsqrt(deg)
    h = jnp.dot(x_ref[...], w1_ref[...], preferred_element_type=jnp.float32)
    h1_ref[...] = h
    hs1_ref[...] = (h * dinv[:, None]).astype(jnp.bfloat16)
    dinv_ref[...] = dinv


def _tc_b_body(tp_ref, h1_ref, dinv_ref, b1_ref, w2_ref,
               h1r_ref, h2_ref, hs2_ref):
    dinv = dinv_ref[...]
    t = tp_ref[0] + tp_ref[1]
    agg = dinv[:, None] * t + (dinv * dinv)[:, None] * h1_ref[...] \
        + b1_ref[...][None, :]
    h1r = jnp.maximum(agg, 0.0)
    h1r_ref[...] = h1r
    h2 = jnp.dot(h1r, w2_ref[...], preferred_element_type=jnp.float32)
    h2_ref[...] = h2
    hs2_ref[...] = (h2 * dinv[:, None]).astype(jnp.bfloat16)


def _tc_c_body(tp_ref, h2_ref, h1r_ref, dinv_ref, b2_ref, batf_ref,
               wc_ref, bc_ref, out_ref, sums_ref):
    b = pl.program_id(0)

    @pl.when(b == 0)
    def _():
        sums_ref[...] = jnp.zeros_like(sums_ref)

    dinv = dinv_ref[...]
    t = tp_ref[0] + tp_ref[1]
    agg = dinv[:, None] * t + (dinv * dinv)[:, None] * h2_ref[...] \
        + b2_ref[...][None, :]
    h2r = jnp.maximum(agg, 0.0)
    ones = jnp.ones((1024, 128), jnp.float32)
    emb = jnp.concatenate([h1r_ref[...], h2r, ones], axis=1)
    gid = lax.broadcasted_iota(jnp.int32, (1024, 128), 1).astype(jnp.float32)
    onehot = (batf_ref[...][:, None] == gid).astype(jnp.float32)
    sums_ref[...] += lax.dot_general(
        onehot, emb, (((0,), (0,)), ((), ())),
        preferred_element_type=jnp.float32)

    @pl.when(b == _GRID - 1)
    def _():
        cnt = jnp.maximum(sums_ref[:, 256:257], 1.0)
        ge = sums_ref[:, :256] / cnt
        out_ref[...] = jnp.dot(ge[:64], wc_ref[...],
                               preferred_element_type=jnp.float32) \
            + bc_ref[...][None, :]


def _row_spec(shape_last):
    return pl.BlockSpec((1024,) + shape_last,
                        lambda b: (b,) + (0,) * len(shape_last))


_tc_a = pl.pallas_call(
    _tc_a_body,
    grid=(_GRID,),
    in_specs=[
        _row_spec((D,)),
        pl.BlockSpec((D, D), lambda b: (0, 0)),
        pl.BlockSpec((NT, 1024), lambda b: (0, b)),
    ],
    out_specs=[_row_spec((D,)), _row_spec((D,)), _row_spec(())],
    out_shape=[
        jax.ShapeDtypeStruct((NP, D), jnp.float32),
        jax.ShapeDtypeStruct((NP, D), jnp.bfloat16),
        jax.ShapeDtypeStruct((NP,), jnp.float32),
    ],
)

_tc_b = pl.pallas_call(
    _tc_b_body,
    grid=(_GRID,),
    in_specs=[
        pl.BlockSpec((2, 1024, D), lambda b: (0, b, 0)),
        _row_spec((D,)),
        _row_spec(()),
        pl.BlockSpec((D,), lambda b: (0,)),
        pl.BlockSpec((D, D), lambda b: (0, 0)),
    ],
    out_specs=[_row_spec((D,)), _row_spec((D,)), _row_spec((D,))],
    out_shape=[
        jax.ShapeDtypeStruct((NP, D), jnp.float32),
        jax.ShapeDtypeStruct((NP, D), jnp.float32),
        jax.ShapeDtypeStruct((NP, D), jnp.bfloat16),
    ],
)

_tc_c = pl.pallas_call(
    _tc_c_body,
    grid=(_GRID,),
    in_specs=[
        pl.BlockSpec((2, 1024, D), lambda b: (0, b, 0)),
        _row_spec((D,)),
        _row_spec((D,)),
        _row_spec(()),
        pl.BlockSpec((D,), lambda b: (0,)),
        _row_spec(()),
        pl.BlockSpec((256, 128), lambda b: (0, 0)),
        pl.BlockSpec((128,), lambda b: (0,)),
    ],
    out_specs=pl.BlockSpec((64, 128), lambda b: (0, 0)),
    out_shape=jax.ShapeDtypeStruct((64, 128), jnp.float32),
    scratch_shapes=[pltpu.VMEM((128, 384), jnp.float32)],
)


@jax.jit
def kernel(x, edge_index, edge_weights, batch, W1, b1, W2, b2, Wc, bc):
    src = edge_index[0]
    dst = edge_index[1]
    pad_e = EP - N_EDGES
    sd = jnp.concatenate(
        [(src << 14) | dst, jnp.zeros((pad_e,), jnp.int32)])
    w_f = jnp.concatenate(
        [edge_weights, jnp.zeros((pad_e,), jnp.float32)])

    def _split(a):
        p0 = jnp.pad(a[:E0].reshape(16, CPT0, CH),
                     ((0, 0), (0, CPTC - CPT0), (0, 0)))
        p1 = jnp.pad(a[E0:].reshape(16, CPT1, CH),
                     ((0, 0), (0, CPTC - CPT1), (0, 0)))
        return jnp.concatenate([p0, p1], axis=0)

    sd_p = _split(sd)
    w_p = _split(w_f)
    x_p = jnp.pad(x, ((0, NP - N_NODES), (0, 0)))
    batf = jnp.concatenate(
        [batch.astype(jnp.float32),
         jnp.full((NP - N_NODES,), 64.0, jnp.float32)])
    wc_p = jnp.pad(Wc, ((0, 0), (0, 128 - Wc.shape[1])))
    bc_p = jnp.pad(bc, ((0, 128 - bc.shape[0]),))

    def _ilv(a):
        return a.reshape(NP, D // 32, 2, 16).swapaxes(2, 3).reshape(NP, D)

    degp = _deg_kernel(sd_p, w_p)
    h1, hs1, dinv = _tc_a(x_p, W1, degp)
    t1 = _msgpass_kernel(_ilv(hs1), sd_p, w_p)
    h1r, h2, hs2 = _tc_b(t1, h1, dinv, b1, W2)
    t2 = _msgpass_kernel(_ilv(hs2), sd_p, w_p)
    logits_p = _tc_c(t2, h2, h1r, dinv, b2, batf, wc_p, bc_p)
    return logits_p[:, :16]


# speedup vs baseline: 1.0218x; 1.0218x over previous
import functools

import jax
import jax.numpy as jnp
from jax import lax
from jax.experimental import pallas as pl
from jax.experimental.pallas import tpu as pltpu
from jax.experimental.pallas import tpu_sc as plsc

N_NODES = 10000
N_EDGES = 320000
D = 128

NP = 10240
NT = 32
CH = 128
CPT0 = 90
CPT1 = 67
CPTC = 90
E0 = 16 * CPT0 * CH
E1 = 16 * CPT1 * CH
EP = E0 + E1
RPT = NP // 16

_mesh = plsc.VectorSubcoreMesh(core_axis_name="c", subcore_axis_name="s")
_sc_params = pltpu.CompilerParams(
    needs_layout_passes=False, use_tc_tiling_on_sc=False)


@functools.partial(
    pl.kernel,
    out_type=jax.ShapeDtypeStruct((NT, NP), jnp.float32),
    mesh=_mesh,
    scratch_types=[
        pltpu.VMEM((CPTC, CH), jnp.int32),
        pltpu.VMEM((CPTC, CH), jnp.float32),
        pltpu.VMEM((NP,), jnp.float32),
    ],
    compiler_params=_sc_params,
)
def _deg_kernel(sd_hbm, w_hbm, out_hbm, sd_v, w_v, deg_v):
    c = lax.axis_index("c")
    s = lax.axis_index("s")
    wid = c * 16 + s
    my_cpt = jnp.where(c == 0, CPT0, CPT1)
    pltpu.sync_copy(sd_hbm.at[wid], sd_v)
    pltpu.sync_copy(w_hbm.at[wid], w_v)

    zero = jnp.zeros((16,), jnp.float32)

    @pl.loop(0, NP // 16)
    def _(i):
        deg_v[pl.ds(i * 16, 16)] = zero

    @pl.loop(0, my_cpt)
    def _(j):
        for q in range(CH // 16):
            idx = jnp.bitwise_and(sd_v[j, pl.ds(q * 16, 16)], 16383)
            val = w_v[j, pl.ds(q * 16, 16)]
            plsc.addupdate_scatter(deg_v, [idx], val)

    pltpu.sync_copy(deg_v, out_hbm.at[wid])


@functools.partial(
    pl.kernel,
    out_type=jax.ShapeDtypeStruct((2, NP, D), jnp.float32),
    mesh=_mesh,
    scratch_types=[
        pltpu.VMEM((CPTC, CH), jnp.int32),
        pltpu.VMEM((CPTC, CH), jnp.float32),
        pltpu.VMEM((CH,), jnp.int32),
        pltpu.VMEM((CH,), jnp.int32),
        pltpu.VMEM((CH, D), jnp.float32),
        pltpu.VMEM_SHARED((NP, D), jnp.float32),
        pltpu.SemaphoreType.DMA,
    ],
    compiler_params=_sc_params,
)
def _msgpass_kernel(hs_hbm, sd_hbm, w_hbm, out_hbm,
                    sd_v, w_v, srci_v, dsti_v, rows_v, acc_sh, gsem):
    c = lax.axis_index("c")
    s = lax.axis_index("s")
    wid = c * 16 + s
    my_cpt = jnp.where(c == 0, CPT0, CPT1)
    pltpu.sync_copy(sd_hbm.at[wid], sd_v)
    pltpu.sync_copy(w_hbm.at[wid], w_v)

    zero = jnp.zeros((16,), jnp.float32)

    @pl.loop(0, CH)
    def _(r):
        for q in range(D // 16):
            rows_v[r, pl.ds(q * 16, 16)] = zero

    base = s * RPT
    for k in range(RPT // CH):
        pltpu.sync_copy(rows_v, acc_sh.at[pl.ds(base + k * CH, CH)])
    plsc.subcore_barrier()

    @pl.loop(0, my_cpt)
    def _(j):
        @plsc.parallel_loop(0, CH // 16)
        def _(q):
            sd = sd_v[j, pl.ds(q * 16, 16)]
            srci_v[pl.ds(q * 16, 16)] = lax.shift_right_logical(sd, 14)
            dsti_v[pl.ds(q * 16, 16)] = jnp.bitwise_and(sd, 16383)

        pltpu.async_copy(hs_hbm.at[srci_v], rows_v, gsem).wait()

        @plsc.parallel_loop(0, CH // 16)
        def _(g):
            wvec = w_v[j, pl.ds(g * 16, 16)]
            for r in range(16):
                wv = wvec[r]
                row = g * 16 + r
                for q in range(D // 16):
                    rows_v[row, pl.ds(q * 16, 16)] = (
                        rows_v[row, pl.ds(q * 16, 16)] * wv)

        pltpu.sync_copy(rows_v, acc_sh.at[dsti_v], add=True)

    plsc.subcore_barrier()
    pltpu.sync_copy(acc_sh.at[pl.ds(s * RPT, RPT)],
                    out_hbm.at[c, pl.ds(s * RPT, RPT)])


_GRID = NP // 2048


def _tc_a_body(x_ref, w1_ref, degp_ref, h1_ref, hs1_ref, dinv_ref):
    deg = 1.0 + jnp.sum(degp_ref[...], axis=0)
    dinv = lax.rsqrt(deg)
    h = jnp.dot(x_ref[...], w1_ref[...], preferred_element_type=jnp.float32)
    h1_ref[...] = h
    hs1_ref[...] = h * dinv[:, None]
    dinv_ref[...] = dinv


def _tc_b_body(tp_ref, h1_ref, dinv_ref, b1_ref, w2_ref,
               h1r_ref, h2_ref, hs2_ref):
    dinv = dinv_ref[...]
    t = tp_ref[0] + tp_ref[1]
    agg = dinv[:, None] * t + (dinv * dinv)[:, None] * h1_ref[...] \
        + b1_ref[...][None, :]
    h1r = jnp.maximum(agg, 0.0)
    h1r_ref[...] = h1r
    h2 = jnp.dot(h1r, w2_ref[...], preferred_element_type=jnp.float32)
    h2_ref[...] = h2
    hs2_ref[...] = h2 * dinv[:, None]


def _tc_c_body(tp_ref, h2_ref, h1r_ref, dinv_ref, b2_ref, batf_ref,
               wc_ref, bc_ref, out_ref, sums_ref):
    b = pl.program_id(0)

    @pl.when(b == 0)
    def _():
        sums_ref[...] = jnp.zeros_like(sums_ref)

    dinv = dinv_ref[...]
    t = tp_ref[0] + tp_ref[1]
    agg = dinv[:, None] * t + (dinv * dinv)[:, None] * h2_ref[...] \
        + b2_ref[...][None, :]
    h2r = jnp.maximum(agg, 0.0)
    ones = jnp.ones((2048, 128), jnp.float32)
    emb = jnp.concatenate([h1r_ref[...], h2r, ones], axis=1)
    gid = lax.broadcasted_iota(jnp.int32, (2048, 128), 1).astype(jnp.float32)
    onehot = (batf_ref[...][:, None] == gid).astype(jnp.float32)
    sums_ref[...] += lax.dot_general(
        onehot, emb, (((0,), (0,)), ((), ())),
        preferred_element_type=jnp.float32)

    @pl.when(b == _GRID - 1)
    def _():
        cnt = jnp.maximum(sums_ref[:, 256:257], 1.0)
        ge = sums_ref[:, :256] / cnt
        out_ref[...] = jnp.dot(ge[:64], wc_ref[...],
                               preferred_element_type=jnp.float32) \
            + bc_ref[...][None, :]


def _row_spec(shape_last):
    return pl.BlockSpec((2048,) + shape_last,
                        lambda b: (b,) + (0,) * len(shape_last))


_tc_a = pl.pallas_call(
    _tc_a_body,
    grid=(_GRID,),
    in_specs=[
        _row_spec((D,)),
        pl.BlockSpec((D, D), lambda b: (0, 0)),
        pl.BlockSpec((NT, 2048), lambda b: (0, b)),
    ],
    out_specs=[_row_spec((D,)), _row_spec((D,)), _row_spec(())],
    out_shape=[
        jax.ShapeDtypeStruct((NP, D), jnp.float32),
        jax.ShapeDtypeStruct((NP, D), jnp.float32),
        jax.ShapeDtypeStruct((NP,), jnp.float32),
    ],
)

_tc_b = pl.pallas_call(
    _tc_b_body,
    grid=(_GRID,),
    in_specs=[
        pl.BlockSpec((2, 2048, D), lambda b: (0, b, 0)),
        _row_spec((D,)),
        _row_spec(()),
        pl.BlockSpec((D,), lambda b: (0,)),
        pl.BlockSpec((D, D), lambda b: (0, 0)),
    ],
    out_specs=[_row_spec((D,)), _row_spec((D,)), _row_spec((D,))],
    out_shape=[
        jax.ShapeDtypeStruct((NP, D), jnp.float32),
        jax.ShapeDtypeStruct((NP, D), jnp.float32),
        jax.ShapeDtypeStruct((NP, D), jnp.float32),
    ],
)

_tc_c = pl.pallas_call(
    _tc_c_body,
    grid=(_GRID,),
    in_specs=[
        pl.BlockSpec((2, 2048, D), lambda b: (0, b, 0)),
        _row_spec((D,)),
        _row_spec((D,)),
        _row_spec(()),
        pl.BlockSpec((D,), lambda b: (0,)),
        _row_spec(()),
        pl.BlockSpec((256, 128), lambda b: (0, 0)),
        pl.BlockSpec((128,), lambda b: (0,)),
    ],
    out_specs=pl.BlockSpec((64, 128), lambda b: (0, 0)),
    out_shape=jax.ShapeDtypeStruct((64, 128), jnp.float32),
    scratch_shapes=[pltpu.VMEM((128, 384), jnp.float32)],
)


@jax.jit
def kernel(x, edge_index, edge_weights, batch, W1, b1, W2, b2, Wc, bc):
    src = edge_index[0]
    dst = edge_index[1]
    pad_e = EP - N_EDGES
    sd = jnp.concatenate(
        [(src << 14) | dst, jnp.zeros((pad_e,), jnp.int32)])
    w_f = jnp.concatenate(
        [edge_weights, jnp.zeros((pad_e,), jnp.float32)])

    def _split(a):
        p0 = jnp.pad(a[:E0].reshape(16, CPT0, CH),
                     ((0, 0), (0, CPTC - CPT0), (0, 0)))
        p1 = jnp.pad(a[E0:].reshape(16, CPT1, CH),
                     ((0, 0), (0, CPTC - CPT1), (0, 0)))
        return jnp.concatenate([p0, p1], axis=0)

    sd_p = _split(sd)
    w_p = _split(w_f)
    x_p = jnp.pad(x, ((0, NP - N_NODES), (0, 0)))
    batf = jnp.concatenate(
        [batch.astype(jnp.float32),
         jnp.full((NP - N_NODES,), 64.0, jnp.float32)])
    wc_p = jnp.pad(Wc, ((0, 0), (0, 128 - Wc.shape[1])))
    bc_p = jnp.pad(bc, ((0, 128 - bc.shape[0]),))

    degp = _deg_kernel(sd_p, w_p)
    h1, hs1, dinv = _tc_a(x_p, W1, degp)
    t1 = _msgpass_kernel(hs1, sd_p, w_p)
    h1r, h2, hs2 = _tc_b(t1, h1, dinv, b1, W2)
    t2 = _msgpass_kernel(hs2, sd_p, w_p)
    logits_p = _tc_c(t2, h2, h1r, dinv, b2, batf, wc_p, bc_p)
    return logits_p[:, :16]
